# baseline (speedup 1.0000x reference)
import jax
import jax.numpy as jnp
from jax.experimental import pallas as pl

TOK = 2048
D = 1024
G = 8
S = 2
NF = 64
FPG = 8
DF = 64
DR = 128
DH = 256
E = G * S
GROUP_TOP_K = 4
TEMP = 1.0

TILE = 256


def _split(x):
    hi = x.astype(jnp.bfloat16)
    lo = (x - hi.astype(jnp.float32)).astype(jnp.bfloat16)
    return hi, lo


def _moe_kernel(hidden_ref, feat_ref, lng_ref, lnb_ref,
                wp_ref, bp_ref, wr1_ref, br1_ref, wr2_ref, br2_ref,
                we_ref, be_ref, w1_ref, b1_ref, bsel_ref, w2_ref, b2_ref,
                out_ref):
    x = hidden_ref[...]

    mu = jnp.mean(x, axis=-1, keepdims=True)
    xc = x - mu
    var = jnp.mean(xc * xc, axis=-1, keepdims=True)
    h = xc * jax.lax.rsqrt(var + 1e-5) * lng_ref[...] + lnb_ref[...]
    hb = h.astype(jnp.bfloat16)

    f_hi, f_lo = _split(feat_ref[...])
    femb = jnp.dot(jnp.concatenate([f_hi, f_lo, f_hi], axis=1), wp_ref[...],
                   preferred_element_type=jnp.float32) + bp_ref[...]
    fe_hi, fe_lo = _split(femb)
    h_hi, h_lo = _split(h)
    rin = jnp.concatenate([h_hi, h_lo, h_hi, fe_hi, fe_lo, fe_hi], axis=1)
    rpre = jnp.dot(rin, wr1_ref[...], preferred_element_type=jnp.float32)
    rh = jax.nn.gelu(rpre + br1_ref[...])
    r_hi, r_lo = _split(rh)
    glogits = (jnp.dot(jnp.concatenate([r_hi, r_lo, r_hi], axis=1),
                       wr2_ref[...], preferred_element_type=jnp.float32)
               + br2_ref[...]) / max(TEMP, 1e-6)

    work = glogits
    neg = jnp.float32(-jnp.inf)
    thr = None
    for _ in range(GROUP_TOP_K):
        thr = jnp.max(work, axis=-1, keepdims=True)
        work = jnp.where(work >= thr, neg, work)
    keep = glogits >= thr
    gmax = jnp.max(glogits, axis=-1, keepdims=True)
    ge = jnp.where(keep, jnp.exp(glogits - gmax), 0.0)
    gw = ge / jnp.sum(ge, axis=-1, keepdims=True)

    elogits = (jnp.dot(hb, we_ref[...], preferred_element_type=jnp.float32)
               + be_ref[...]) / max(TEMP, 1e-6)
    el = elogits.reshape(TILE, G, S)
    em = jnp.max(el, axis=-1, keepdims=True)
    ee = jnp.exp(el - em)
    ew = ee / jnp.sum(ee, axis=-1, keepdims=True)

    cw = (gw[:, :, None] * ew).reshape(TILE, E)
    cwb = jnp.dot(cw.astype(jnp.bfloat16), bsel_ref[...],
                  preferred_element_type=jnp.float32)

    h1 = jnp.dot(hb, w1_ref[...], preferred_element_type=jnp.float32)
    u = (jax.nn.gelu(h1 + b1_ref[...]) * cwb).astype(jnp.bfloat16)
    v = jnp.dot(u, w2_ref[...], preferred_element_type=jnp.float32)
    vb = jnp.dot(cw, b2_ref[...], preferred_element_type=jnp.float32)
    out_ref[...] = x + v + vb


@jax.jit
def kernel(hidden, features, ln_g, ln_b, Wp, bp, Wr1, br1, Wr2, br2,
           We, be, W1, b1, W2, b2, group_idx):
    B = hidden.shape[0]

    onehot = jax.nn.one_hot(group_idx, NF, dtype=Wp.dtype, axis=0)
    wp_full = jnp.einsum('ngf,gfd->ngd', onehot, Wp).reshape(NF, G * DF)

    def split(w):
        hi = w.astype(jnp.bfloat16)
        lo = (w - hi.astype(jnp.float32)).astype(jnp.bfloat16)
        return hi, lo

    def stack3(w):
        hi, lo = split(w)
        return jnp.concatenate([hi, hi, lo], axis=0)

    wp3 = stack3(wp_full)

    wr1h = jnp.transpose(Wr1[:, :D, :], (1, 0, 2)).reshape(D, G * DR)
    wr1f = jnp.zeros((G * DF, G * DR), Wr1.dtype)
    for g in range(G):
        wr1f = wr1f.at[g * DF:(g + 1) * DF, g * DR:(g + 1) * DR].set(
            Wr1[g, D:, :])
    wr1_3 = jnp.concatenate([stack3(wr1h), stack3(wr1f)], axis=0)
    br1_flat = br1.reshape(1, G * DR)
    wr2_bd = jnp.zeros((G * DR, G), Wr2.dtype)
    for g in range(G):
        wr2_bd = wr2_bd.at[g * DR:(g + 1) * DR, g].set(Wr2[g, :, 0])
    wr2_3 = stack3(wr2_bd)
    br2_row = br2.reshape(1, G)

    we_flat = jnp.transpose(We, (1, 0, 2)).reshape(D, E).astype(jnp.bfloat16)
    be_flat = be.reshape(1, E)

    w1cat = jnp.transpose(W1, (1, 0, 2)).reshape(D, E * DH).astype(jnp.bfloat16)
    b1flat = b1.reshape(1, E * DH)
    w2cat = W2.reshape(E * DH, D).astype(jnp.bfloat16)
    bsel = jnp.repeat(jnp.eye(E, dtype=jnp.bfloat16), DH, axis=1)

    n_tiles = B // TILE
    full = lambda shape: pl.BlockSpec(shape, lambda i: (0,) * len(shape))

    out = pl.pallas_call(
        _moe_kernel,
        grid=(n_tiles,),
        in_specs=[
            pl.BlockSpec((TILE, D), lambda i: (i, 0)),
            pl.BlockSpec((TILE, NF), lambda i: (i, 0)),
            full((1, D)), full((1, D)),
            full((3 * NF, G * DF)), full((1, G * DF)),
            full((3 * (D + G * DF), G * DR)), full((1, G * DR)),
            full((3 * G * DR, G)), full((1, G)),
            full((D, E)), full((1, E)),
            full((D, E * DH)), full((1, E * DH)), full((E, E * DH)),
            full((E * DH, D)), full((E, D)),
        ],
        out_specs=pl.BlockSpec((TILE, D), lambda i: (i, 0)),
        out_shape=jax.ShapeDtypeStruct((B, D), jnp.float32),
    )(hidden, features, ln_g.reshape(1, D), ln_b.reshape(1, D),
      wp3, bp.reshape(1, G * DF), wr1_3, br1_flat, wr2_3, br2_row,
      we_flat, be_flat, w1cat, b1flat, bsel, w2cat, b2)
    return out


# speedup vs baseline: 1.0191x; 1.0191x over previous
import jax
import jax.numpy as jnp
from jax.experimental import pallas as pl

TOK = 2048
D = 1024
G = 8
S = 2
NF = 64
FPG = 8
DF = 64
DR = 128
DH = 256
E = G * S
GROUP_TOP_K = 4
TEMP = 1.0

TILE = 256


def _bf(x):
    return x.astype(jnp.bfloat16)


def _moe_kernel(hidden_ref, feat_ref, lng_ref, lnb_ref,
                wp_ref, bp_ref, wr1h_ref, wr1f_ref, br1_ref, wr2_ref, br2_ref,
                we_ref, be_ref, w1_ref, b1_ref, bsel_ref, w2_ref, b2_ref,
                out_ref):
    x = hidden_ref[...]

    mu = jnp.mean(x, axis=-1, keepdims=True)
    xc = x - mu
    var = jnp.mean(xc * xc, axis=-1, keepdims=True)
    h = xc / jnp.sqrt(var + 1e-5) * lng_ref[...] + lnb_ref[...]
    hb = _bf(h)

    femb = jnp.dot(_bf(feat_ref[...]), wp_ref[...],
                   preferred_element_type=jnp.float32) + bp_ref[...]

    rpre = jnp.dot(hb, wr1h_ref[...], preferred_element_type=jnp.float32)
    rpre += jnp.dot(_bf(femb), wr1f_ref[...], preferred_element_type=jnp.float32)
    rh = jax.nn.gelu(rpre + br1_ref[...])
    glogits = (jnp.dot(_bf(rh), wr2_ref[...], preferred_element_type=jnp.float32)
               + br2_ref[...]) / max(TEMP, 1e-6)

    work = glogits
    neg = jnp.float32(-jnp.inf)
    thr = None
    for _ in range(GROUP_TOP_K):
        thr = jnp.max(work, axis=-1, keepdims=True)
        work = jnp.where(work >= thr, neg, work)
    keep = glogits >= thr
    gmax = jnp.max(glogits, axis=-1, keepdims=True)
    ge = jnp.where(keep, jnp.exp(glogits - gmax), 0.0)
    gw = ge / jnp.sum(ge, axis=-1, keepdims=True)

    elogits = (jnp.dot(hb, we_ref[...], preferred_element_type=jnp.float32)
               + be_ref[...]) / max(TEMP, 1e-6)
    el = elogits.reshape(TILE, G, S)
    em = jnp.max(el, axis=-1, keepdims=True)
    ee = jnp.exp(el - em)
    ew = ee / jnp.sum(ee, axis=-1, keepdims=True)

    cw = (gw[:, :, None] * ew).reshape(TILE, E)
    cwb = jnp.dot(_bf(cw), bsel_ref[...],
                  preferred_element_type=jnp.float32)

    h1 = jnp.dot(hb, w1_ref[...], preferred_element_type=jnp.float32)
    u = _bf(jax.nn.gelu(h1 + b1_ref[...]) * cwb)
    v = jnp.dot(u, w2_ref[...], preferred_element_type=jnp.float32)
    vb = jnp.dot(cw, b2_ref[...], preferred_element_type=jnp.float32)
    out_ref[...] = x + v + vb


@jax.jit
def kernel(hidden, features, ln_g, ln_b, Wp, bp, Wr1, br1, Wr2, br2,
           We, be, W1, b1, W2, b2, group_idx):
    B = hidden.shape[0]

    onehot = jax.nn.one_hot(group_idx, NF, dtype=Wp.dtype, axis=0)
    wp_full = jnp.einsum('ngf,gfd->ngd', onehot, Wp,
                         precision=jax.lax.Precision.HIGHEST).reshape(NF, G * DF)

    wr1h = jnp.transpose(Wr1[:, :D, :], (1, 0, 2)).reshape(D, G * DR)
    wr1f = jnp.zeros((G * DF, G * DR), Wr1.dtype)
    for g in range(G):
        wr1f = wr1f.at[g * DF:(g + 1) * DF, g * DR:(g + 1) * DR].set(
            Wr1[g, D:, :])
    br1_flat = br1.reshape(1, G * DR)
    wr2_bd = jnp.zeros((G * DR, G), Wr2.dtype)
    for g in range(G):
        wr2_bd = wr2_bd.at[g * DR:(g + 1) * DR, g].set(Wr2[g, :, 0])
    br2_row = br2.reshape(1, G)

    we_flat = jnp.transpose(We, (1, 0, 2)).reshape(D, E)
    be_flat = be.reshape(1, E)

    w1cat = jnp.transpose(W1, (1, 0, 2)).reshape(D, E * DH)
    b1flat = b1.reshape(1, E * DH)
    w2cat = W2.reshape(E * DH, D)
    bsel = jnp.repeat(jnp.eye(E, dtype=jnp.bfloat16), DH, axis=1)

    bf = jnp.bfloat16
    n_tiles = B // TILE
    full = lambda shape: pl.BlockSpec(shape, lambda i: (0,) * len(shape))

    out = pl.pallas_call(
        _moe_kernel,
        grid=(n_tiles,),
        in_specs=[
            pl.BlockSpec((TILE, D), lambda i: (i, 0)),
            pl.BlockSpec((TILE, NF), lambda i: (i, 0)),
            full((1, D)), full((1, D)),
            full((NF, G * DF)), full((1, G * DF)),
            full((D, G * DR)), full((G * DF, G * DR)), full((1, G * DR)),
            full((G * DR, G)), full((1, G)),
            full((D, E)), full((1, E)),
            full((D, E * DH)), full((1, E * DH)), full((E, E * DH)),
            full((E * DH, D)), full((E, D)),
        ],
        out_specs=pl.BlockSpec((TILE, D), lambda i: (i, 0)),
        out_shape=jax.ShapeDtypeStruct((B, D), jnp.float32),
    )(hidden, features, ln_g.reshape(1, D), ln_b.reshape(1, D),
      wp_full.astype(bf), bp.reshape(1, G * DF), wr1h.astype(bf),
      wr1f.astype(bf), br1_flat, wr2_bd.astype(bf), br2_row,
      we_flat.astype(bf), be_flat, w1cat.astype(bf), b1flat, bsel,
      w2cat.astype(bf), b2)
    return out


# speedup vs baseline: 1.4792x; 1.4515x over previous
import jax
import jax.numpy as jnp
from jax.experimental import pallas as pl

TOK = 2048
D = 1024
G = 8
S = 2
NF = 64
FPG = 8
DF = 64
DR = 128
DH = 256
E = G * S
GROUP_TOP_K = 4
TEMP = 1.0

TILE = 256


def _bf(x):
    return x.astype(jnp.bfloat16)


def _moe_kernel(hidden_ref, feat_ref, lng_ref, lnb_ref,
                wp_ref, bp_ref, wr1h_ref, wr1f_ref, br1_ref, wr2_ref, br2_ref,
                we_ref, be_ref, w1_ref, b1_ref, w2_ref, b2_ref,
                out_ref):
    x = hidden_ref[...]

    mu = jnp.mean(x, axis=-1, keepdims=True)
    xc = x - mu
    var = jnp.mean(xc * xc, axis=-1, keepdims=True)
    h = xc / jnp.sqrt(var + 1e-5) * lng_ref[...] + lnb_ref[...]
    hb = _bf(h)

    femb = jnp.dot(_bf(feat_ref[...]), wp_ref[...],
                   preferred_element_type=jnp.float32) + bp_ref[...]

    rpre = jnp.dot(hb, wr1h_ref[...], preferred_element_type=jnp.float32)
    rpre += jnp.dot(_bf(femb), wr1f_ref[...], preferred_element_type=jnp.float32)
    rh = jax.nn.gelu(rpre + br1_ref[...])
    glogits = (jnp.dot(_bf(rh), wr2_ref[...], preferred_element_type=jnp.float32)
               + br2_ref[...]) / max(TEMP, 1e-6)

    work = glogits
    neg = jnp.float32(-jnp.inf)
    thr = None
    for _ in range(GROUP_TOP_K):
        thr = jnp.max(work, axis=-1, keepdims=True)
        work = jnp.where(work >= thr, neg, work)
    keep = glogits >= thr
    gmax = jnp.max(glogits, axis=-1, keepdims=True)
    ge = jnp.where(keep, jnp.exp(glogits - gmax), 0.0)
    gw = ge / jnp.sum(ge, axis=-1, keepdims=True)

    elogits = (jnp.dot(hb, we_ref[...], preferred_element_type=jnp.float32)
               + be_ref[...]) / max(TEMP, 1e-6)
    el = elogits.reshape(TILE, G, S)
    em = jnp.max(el, axis=-1, keepdims=True)
    ee = jnp.exp(el - em)
    ew = ee / jnp.sum(ee, axis=-1, keepdims=True)

    cw = (gw[:, :, None] * ew).reshape(TILE, E)

    ublocks = []
    for e in range(E):
        ue = jnp.dot(hb, w1_ref[e], preferred_element_type=jnp.float32)
        ue = jax.nn.gelu(ue + b1_ref[e][None]) * cw[:, e][:, None]
        ublocks.append(_bf(ue))
    u = jnp.concatenate(ublocks, axis=1)
    v = jnp.dot(u, w2_ref[...], preferred_element_type=jnp.float32)
    vb = jnp.dot(cw, b2_ref[...], preferred_element_type=jnp.float32)
    out_ref[...] = x + v + vb


@jax.jit
def kernel(hidden, features, ln_g, ln_b, Wp, bp, Wr1, br1, Wr2, br2,
           We, be, W1, b1, W2, b2, group_idx):
    B = hidden.shape[0]

    onehot = jax.nn.one_hot(group_idx, NF, dtype=Wp.dtype, axis=0)
    wp_full = jnp.einsum('ngf,gfd->ngd', onehot, Wp,
                         precision=jax.lax.Precision.HIGHEST).reshape(NF, G * DF)

    wr1h = jnp.transpose(Wr1[:, :D, :], (1, 0, 2)).reshape(D, G * DR)
    wr1f = jnp.zeros((G * DF, G * DR), Wr1.dtype)
    for g in range(G):
        wr1f = wr1f.at[g * DF:(g + 1) * DF, g * DR:(g + 1) * DR].set(
            Wr1[g, D:, :])
    br1_flat = br1.reshape(1, G * DR)
    wr2_bd = jnp.zeros((G * DR, G), Wr2.dtype)
    for g in range(G):
        wr2_bd = wr2_bd.at[g * DR:(g + 1) * DR, g].set(Wr2[g, :, 0])
    br2_row = br2.reshape(1, G)

    we_flat = jnp.transpose(We, (1, 0, 2)).reshape(D, E)
    be_flat = be.reshape(1, E)

    w1b = W1.astype(jnp.bfloat16)
    w2cat = W2.reshape(E * DH, D).astype(jnp.bfloat16)

    bf = jnp.bfloat16
    n_tiles = B // TILE
    full = lambda shape: pl.BlockSpec(shape, lambda i: (0,) * len(shape))

    out = pl.pallas_call(
        _moe_kernel,
        grid=(n_tiles,),
        in_specs=[
            pl.BlockSpec((TILE, D), lambda i: (i, 0)),
            pl.BlockSpec((TILE, NF), lambda i: (i, 0)),
            full((1, D)), full((1, D)),
            full((NF, G * DF)), full((1, G * DF)),
            full((D, G * DR)), full((G * DF, G * DR)), full((1, G * DR)),
            full((G * DR, G)), full((1, G)),
            full((D, E)), full((1, E)),
            full((E, D, DH)), full((E, DH)),
            full((E * DH, D)), full((E, D)),
        ],
        out_specs=pl.BlockSpec((TILE, D), lambda i: (i, 0)),
        out_shape=jax.ShapeDtypeStruct((B, D), jnp.float32),
    )(hidden, features, ln_g.reshape(1, D), ln_b.reshape(1, D),
      wp_full.astype(bf), bp.reshape(1, G * DF), wr1h.astype(bf),
      wr1f.astype(bf), br1_flat, wr2_bd.astype(bf), br2_row,
      we_flat.astype(bf), be_flat, w1b, b1, w2cat, b2)
    return out


# speedup vs baseline: 1.5065x; 1.0185x over previous
import jax
import jax.numpy as jnp
from jax.experimental import pallas as pl

TOK = 2048
D = 1024
G = 8
S = 2
NF = 64
FPG = 8
DF = 64
DR = 128
DH = 256
E = G * S
GROUP_TOP_K = 4
TEMP = 1.0

TILE = 512


def _bf(x):
    return x.astype(jnp.bfloat16)


def _moe_kernel(hidden_ref, feat_ref, lng_ref, lnb_ref,
                wp_ref, bp_ref, wr1h_ref, wr1f_ref, br1_ref, wr2_ref, br2_ref,
                we_ref, be_ref, w1_ref, b1_ref, w2_ref, b2_ref,
                out_ref):
    x = hidden_ref[...]

    mu = jnp.mean(x, axis=-1, keepdims=True)
    xc = x - mu
    var = jnp.mean(xc * xc, axis=-1, keepdims=True)
    h = xc / jnp.sqrt(var + 1e-5) * lng_ref[...] + lnb_ref[...]
    hb = _bf(h)

    femb = jnp.dot(_bf(feat_ref[...]), wp_ref[...],
                   preferred_element_type=jnp.float32) + bp_ref[...]

    rpre = jnp.dot(hb, wr1h_ref[...], preferred_element_type=jnp.float32)
    rpre += jnp.dot(_bf(femb), wr1f_ref[...], preferred_element_type=jnp.float32)
    rh = jax.nn.gelu(rpre + br1_ref[...])
    glogits = (jnp.dot(_bf(rh), wr2_ref[...], preferred_element_type=jnp.float32)
               + br2_ref[...]) / max(TEMP, 1e-6)

    work = glogits
    neg = jnp.float32(-jnp.inf)
    thr = None
    for _ in range(GROUP_TOP_K):
        thr = jnp.max(work, axis=-1, keepdims=True)
        work = jnp.where(work >= thr, neg, work)
    keep = glogits >= thr
    gmax = jnp.max(glogits, axis=-1, keepdims=True)
    ge = jnp.where(keep, jnp.exp(glogits - gmax), 0.0)
    gw = ge / jnp.sum(ge, axis=-1, keepdims=True)

    elogits = (jnp.dot(hb, we_ref[...], preferred_element_type=jnp.float32)
               + be_ref[...]) / max(TEMP, 1e-6)
    el = elogits.reshape(TILE, G, S)
    em = jnp.max(el, axis=-1, keepdims=True)
    ee = jnp.exp(el - em)
    ew = ee / jnp.sum(ee, axis=-1, keepdims=True)

    cw = (gw[:, :, None] * ew).reshape(TILE, E)

    ublocks = []
    for e in range(E):
        ue = jnp.dot(hb, w1_ref[e], preferred_element_type=jnp.float32)
        ue = jax.nn.gelu(ue + b1_ref[e][None]) * cw[:, e][:, None]
        ublocks.append(_bf(ue))
    u = jnp.concatenate(ublocks, axis=1)
    v = jnp.dot(u, w2_ref[...], preferred_element_type=jnp.float32)
    vb = jnp.dot(cw, b2_ref[...], preferred_element_type=jnp.float32)
    out_ref[...] = x + v + vb


@jax.jit
def kernel(hidden, features, ln_g, ln_b, Wp, bp, Wr1, br1, Wr2, br2,
           We, be, W1, b1, W2, b2, group_idx):
    B = hidden.shape[0]

    onehot = jax.nn.one_hot(group_idx, NF, dtype=Wp.dtype, axis=0)
    wp_full = jnp.einsum('ngf,gfd->ngd', onehot, Wp,
                         precision=jax.lax.Precision.HIGHEST).reshape(NF, G * DF)

    wr1h = jnp.transpose(Wr1[:, :D, :], (1, 0, 2)).reshape(D, G * DR)
    wr1f = jnp.zeros((G * DF, G * DR), Wr1.dtype)
    for g in range(G):
        wr1f = wr1f.at[g * DF:(g + 1) * DF, g * DR:(g + 1) * DR].set(
            Wr1[g, D:, :])
    br1_flat = br1.reshape(1, G * DR)
    wr2_bd = jnp.zeros((G * DR, G), Wr2.dtype)
    for g in range(G):
        wr2_bd = wr2_bd.at[g * DR:(g + 1) * DR, g].set(Wr2[g, :, 0])
    br2_row = br2.reshape(1, G)

    we_flat = jnp.transpose(We, (1, 0, 2)).reshape(D, E)
    be_flat = be.reshape(1, E)

    w1b = W1.astype(jnp.bfloat16)
    w2cat = W2.reshape(E * DH, D).astype(jnp.bfloat16)

    bf = jnp.bfloat16
    n_tiles = B // TILE
    full = lambda shape: pl.BlockSpec(shape, lambda i: (0,) * len(shape))

    out = pl.pallas_call(
        _moe_kernel,
        grid=(n_tiles,),
        in_specs=[
            pl.BlockSpec((TILE, D), lambda i: (i, 0)),
            pl.BlockSpec((TILE, NF), lambda i: (i, 0)),
            full((1, D)), full((1, D)),
            full((NF, G * DF)), full((1, G * DF)),
            full((D, G * DR)), full((G * DF, G * DR)), full((1, G * DR)),
            full((G * DR, G)), full((1, G)),
            full((D, E)), full((1, E)),
            full((E, D, DH)), full((E, DH)),
            full((E * DH, D)), full((E, D)),
        ],
        out_specs=pl.BlockSpec((TILE, D), lambda i: (i, 0)),
        out_shape=jax.ShapeDtypeStruct((B, D), jnp.float32),
    )(hidden, features, ln_g.reshape(1, D), ln_b.reshape(1, D),
      wp_full.astype(bf), bp.reshape(1, G * DF), wr1h.astype(bf),
      wr1f.astype(bf), br1_flat, wr2_bd.astype(bf), br2_row,
      we_flat.astype(bf), be_flat, w1b, b1, w2cat, b2)
    return out


# speedup vs baseline: 1.5090x; 1.0016x over previous
import jax
import jax.numpy as jnp
from jax.experimental import pallas as pl
from jax.experimental.pallas import tpu as pltpu

TOK = 2048
D = 1024
G = 8
S = 2
NF = 64
FPG = 8
DF = 64
DR = 128
DH = 256
E = G * S
GROUP_TOP_K = 4
TEMP = 1.0

TILE = 512


def _bf(x):
    return x.astype(jnp.bfloat16)


def _moe_kernel(hidden_ref, feat_ref, lng_ref, lnb_ref,
                wp_ref, bp_ref, wr1h_ref, wr1f_ref, br1_ref, wr2_ref, br2_ref,
                we_ref, be_ref, w1_ref, b1_ref, w2_ref, b2_ref,
                out_ref, u_ref):
    x = hidden_ref[...]

    mu = jnp.mean(x, axis=-1, keepdims=True)
    xc = x - mu
    var = jnp.mean(xc * xc, axis=-1, keepdims=True)
    h = xc / jnp.sqrt(var + 1e-5) * lng_ref[...] + lnb_ref[...]
    hb = _bf(h)

    femb = jnp.dot(_bf(feat_ref[...]), wp_ref[...],
                   preferred_element_type=jnp.float32) + bp_ref[...]

    rpre = jnp.dot(hb, wr1h_ref[...], preferred_element_type=jnp.float32)
    rpre += jnp.dot(_bf(femb), wr1f_ref[...], preferred_element_type=jnp.float32)
    rh = jax.nn.gelu(rpre + br1_ref[...])
    glogits = (jnp.dot(_bf(rh), wr2_ref[...], preferred_element_type=jnp.float32)
               + br2_ref[...]) / max(TEMP, 1e-6)

    work = glogits
    neg = jnp.float32(-jnp.inf)
    thr = None
    for _ in range(GROUP_TOP_K):
        thr = jnp.max(work, axis=-1, keepdims=True)
        work = jnp.where(work >= thr, neg, work)
    keep = glogits >= thr
    gmax = jnp.max(glogits, axis=-1, keepdims=True)
    ge = jnp.where(keep, jnp.exp(glogits - gmax), 0.0)
    gw = ge / jnp.sum(ge, axis=-1, keepdims=True)

    elogits = (jnp.dot(hb, we_ref[...], preferred_element_type=jnp.float32)
               + be_ref[...]) / max(TEMP, 1e-6)
    el = elogits.reshape(TILE, G, S)
    em = jnp.max(el, axis=-1, keepdims=True)
    ee = jnp.exp(el - em)
    ew = ee / jnp.sum(ee, axis=-1, keepdims=True)

    cw = (gw[:, :, None] * ew).reshape(TILE, E)

    for e in range(E):
        ue = jnp.dot(hb, w1_ref[e], preferred_element_type=jnp.float32)
        ue = jax.nn.gelu(ue + b1_ref[e][None]) * cw[:, e][:, None]
        u_ref[:, e * DH:(e + 1) * DH] = _bf(ue)
    v = jnp.dot(u_ref[...], w2_ref[...], preferred_element_type=jnp.float32)
    vb = jnp.dot(cw, b2_ref[...], preferred_element_type=jnp.float32)
    out_ref[...] = x + v + vb


@jax.jit
def kernel(hidden, features, ln_g, ln_b, Wp, bp, Wr1, br1, Wr2, br2,
           We, be, W1, b1, W2, b2, group_idx):
    B = hidden.shape[0]

    onehot = jax.nn.one_hot(group_idx, NF, dtype=Wp.dtype, axis=0)
    wp_full = jnp.einsum('ngf,gfd->ngd', onehot, Wp,
                         precision=jax.lax.Precision.HIGHEST).reshape(NF, G * DF)

    wr1h = jnp.transpose(Wr1[:, :D, :], (1, 0, 2)).reshape(D, G * DR)
    wr1f = jnp.zeros((G * DF, G * DR), Wr1.dtype)
    for g in range(G):
        wr1f = wr1f.at[g * DF:(g + 1) * DF, g * DR:(g + 1) * DR].set(
            Wr1[g, D:, :])
    br1_flat = br1.reshape(1, G * DR)
    wr2_bd = jnp.zeros((G * DR, G), Wr2.dtype)
    for g in range(G):
        wr2_bd = wr2_bd.at[g * DR:(g + 1) * DR, g].set(Wr2[g, :, 0])
    br2_row = br2.reshape(1, G)

    we_flat = jnp.transpose(We, (1, 0, 2)).reshape(D, E)
    be_flat = be.reshape(1, E)

    w1b = W1.astype(jnp.bfloat16)
    w2cat = W2.reshape(E * DH, D).astype(jnp.bfloat16)

    bf = jnp.bfloat16
    n_tiles = B // TILE
    full = lambda shape: pl.BlockSpec(shape, lambda i: (0,) * len(shape))

    out = pl.pallas_call(
        _moe_kernel,
        grid=(n_tiles,),
        in_specs=[
            pl.BlockSpec((TILE, D), lambda i: (i, 0)),
            pl.BlockSpec((TILE, NF), lambda i: (i, 0)),
            full((1, D)), full((1, D)),
            full((NF, G * DF)), full((1, G * DF)),
            full((D, G * DR)), full((G * DF, G * DR)), full((1, G * DR)),
            full((G * DR, G)), full((1, G)),
            full((D, E)), full((1, E)),
            full((E, D, DH)), full((E, DH)),
            full((E * DH, D)), full((E, D)),
        ],
        out_specs=pl.BlockSpec((TILE, D), lambda i: (i, 0)),
        out_shape=jax.ShapeDtypeStruct((B, D), jnp.float32),
        scratch_shapes=[pltpu.VMEM((TILE, E * DH), jnp.bfloat16)],
    )(hidden, features, ln_g.reshape(1, D), ln_b.reshape(1, D),
      wp_full.astype(bf), bp.reshape(1, G * DF), wr1h.astype(bf),
      wr1f.astype(bf), br1_flat, wr2_bd.astype(bf), br2_row,
      we_flat.astype(bf), be_flat, w1b, b1, w2cat, b2)
    return out


# speedup vs baseline: 1.5185x; 1.0063x over previous
import jax
import jax.numpy as jnp
from jax.experimental import pallas as pl
from jax.experimental.pallas import tpu as pltpu

TOK = 2048
D = 1024
G = 8
S = 2
NF = 64
FPG = 8
DF = 64
DR = 128
DH = 256
E = G * S
GROUP_TOP_K = 4
TEMP = 1.0

TILE = 512


def _bf(x):
    return x.astype(jnp.bfloat16)


def _moe_kernel(hidden_ref, feat_ref, lng_ref, lnb_ref,
                wp_ref, bp_ref, wr1h_ref, wr1f_ref, br1_ref, wr2_ref, br2_ref,
                we_ref, be_ref, w1_ref, b1_ref, w2_ref, b2_ref,
                out_ref, u_ref):
    x = hidden_ref[...]

    mu = jnp.mean(x, axis=-1, keepdims=True)
    xc = x - mu
    var = jnp.mean(xc * xc, axis=-1, keepdims=True)
    h = xc / jnp.sqrt(var + 1e-5) * lng_ref[...] + lnb_ref[...]

    femb = jnp.dot(feat_ref[...], wp_ref[...],
                   preferred_element_type=jnp.float32) + bp_ref[...]

    rpre = jnp.dot(h, wr1h_ref[...], preferred_element_type=jnp.float32)
    rpre += jnp.dot(femb, wr1f_ref[...], preferred_element_type=jnp.float32)
    rh = jax.nn.gelu(rpre + br1_ref[...])
    glogits = (jnp.dot(rh, wr2_ref[...], preferred_element_type=jnp.float32)
               + br2_ref[...]) / max(TEMP, 1e-6)

    work = glogits
    neg = jnp.float32(-jnp.inf)
    thr = None
    for _ in range(GROUP_TOP_K):
        thr = jnp.max(work, axis=-1, keepdims=True)
        work = jnp.where(work >= thr, neg, work)
    keep = glogits >= thr
    gmax = jnp.max(glogits, axis=-1, keepdims=True)
    ge = jnp.where(keep, jnp.exp(glogits - gmax), 0.0)
    gw = ge / jnp.sum(ge, axis=-1, keepdims=True)

    elogits = (jnp.dot(h, we_ref[...], preferred_element_type=jnp.float32)
               + be_ref[...]) / max(TEMP, 1e-6)
    el = elogits.reshape(TILE, G, S)
    em = jnp.max(el, axis=-1, keepdims=True)
    ee = jnp.exp(el - em)
    ew = ee / jnp.sum(ee, axis=-1, keepdims=True)

    cw = (gw[:, :, None] * ew).reshape(TILE, E)

    for e in range(E):
        ue = jnp.dot(h, w1_ref[e], preferred_element_type=jnp.float32)
        ue = jax.nn.gelu(ue + b1_ref[e][None]) * cw[:, e][:, None]
        u_ref[:, e * DH:(e + 1) * DH] = _bf(ue)
    v = jnp.dot(u_ref[...], w2_ref[...], preferred_element_type=jnp.float32)
    vb = jnp.dot(cw, b2_ref[...], preferred_element_type=jnp.float32)
    out_ref[...] = x + v + vb


@jax.jit
def kernel(hidden, features, ln_g, ln_b, Wp, bp, Wr1, br1, Wr2, br2,
           We, be, W1, b1, W2, b2, group_idx):
    B = hidden.shape[0]

    onehot = jax.nn.one_hot(group_idx, NF, dtype=Wp.dtype, axis=0)
    wp_full = jnp.einsum('ngf,gfd->ngd', onehot, Wp,
                         precision=jax.lax.Precision.HIGHEST).reshape(NF, G * DF)

    wr1h = jnp.transpose(Wr1[:, :D, :], (1, 0, 2)).reshape(D, G * DR)
    wr1f = jnp.zeros((G * DF, G * DR), Wr1.dtype)
    for g in range(G):
        wr1f = wr1f.at[g * DF:(g + 1) * DF, g * DR:(g + 1) * DR].set(
            Wr1[g, D:, :])
    br1_flat = br1.reshape(1, G * DR)
    wr2_bd = jnp.zeros((G * DR, G), Wr2.dtype)
    for g in range(G):
        wr2_bd = wr2_bd.at[g * DR:(g + 1) * DR, g].set(Wr2[g, :, 0])
    br2_row = br2.reshape(1, G)

    we_flat = jnp.transpose(We, (1, 0, 2)).reshape(D, E)
    be_flat = be.reshape(1, E)

    w2cat = W2.reshape(E * DH, D).astype(jnp.bfloat16)

    n_tiles = B // TILE
    full = lambda shape: pl.BlockSpec(shape, lambda i: (0,) * len(shape))

    out = pl.pallas_call(
        _moe_kernel,
        grid=(n_tiles,),
        in_specs=[
            pl.BlockSpec((TILE, D), lambda i: (i, 0)),
            pl.BlockSpec((TILE, NF), lambda i: (i, 0)),
            full((1, D)), full((1, D)),
            full((NF, G * DF)), full((1, G * DF)),
            full((D, G * DR)), full((G * DF, G * DR)), full((1, G * DR)),
            full((G * DR, G)), full((1, G)),
            full((D, E)), full((1, E)),
            full((E, D, DH)), full((E, DH)),
            full((E * DH, D)), full((E, D)),
        ],
        out_specs=pl.BlockSpec((TILE, D), lambda i: (i, 0)),
        out_shape=jax.ShapeDtypeStruct((B, D), jnp.float32),
        scratch_shapes=[pltpu.VMEM((TILE, E * DH), jnp.bfloat16)],
    )(hidden, features, ln_g.reshape(1, D), ln_b.reshape(1, D),
      wp_full, bp.reshape(1, G * DF), wr1h,
      wr1f, br1_flat, wr2_bd, br2_row,
      we_flat, be_flat, W1, b1, w2cat, b2)
    return out
